# baseline (speedup 1.0000x reference)
import functools

import jax
import jax.numpy as jnp
from jax import lax
from jax.experimental import pallas as pl
from jax.experimental.pallas import tpu as pltpu
from jax.experimental.pallas import tpu_sc as plsc

_X_MAX = 100.0
_ALPHA = 0.75
_B = 4096
_NC = 2
_NS = 16
_NW = _NC * _NS
_BPW = _B // _NW
_E = 64
_L = 16


def _sc_gather_dot(w_data, v_data, w_embed, v_embed, w_bias, v_bias):
    mesh = plsc.VectorSubcoreMesh(core_axis_name="c", subcore_axis_name="s")

    @functools.partial(
        pl.kernel,
        mesh=mesh,
        out_type=(
            jax.ShapeDtypeStruct((_B,), jnp.float32),
            jax.ShapeDtypeStruct((_B,), jnp.float32),
        ),
        scratch_types=[
            pltpu.VMEM((_BPW,), jnp.int32),
            pltpu.VMEM((_BPW,), jnp.int32),
            pltpu.VMEM((_BPW, _E), jnp.float32),
            pltpu.VMEM((_BPW, _E), jnp.float32),
            pltpu.VMEM((_BPW,), jnp.float32),
            pltpu.VMEM((_BPW,), jnp.float32),
            pltpu.VMEM((_BPW,), jnp.float32),
            pltpu.VMEM((_BPW,), jnp.float32),
            pltpu.SemaphoreType.DMA,
        ],
        compiler_params=pltpu.CompilerParams(use_tc_tiling_on_sc=False),
    )
    def k(w_data_h, v_data_h, w_embed_h, v_embed_h, w_bias_h, v_bias_h,
          dot_out, s_out, widx, vidx, wrows, vrows, wb, vb, dotv, sv, sem):
        wid = lax.axis_index("s") * _NC + lax.axis_index("c")
        base = wid * _BPW
        pltpu.sync_copy(w_data_h.at[pl.ds(base, _BPW)], widx)
        pltpu.sync_copy(v_data_h.at[pl.ds(base, _BPW)], vidx)
        c1 = pltpu.async_copy(w_embed_h.at[widx], wrows, sem)
        c2 = pltpu.async_copy(v_embed_h.at[vidx], vrows, sem)
        c3 = pltpu.async_copy(w_bias_h.at[widx], wb, sem)
        c4 = pltpu.async_copy(v_bias_h.at[vidx], vb, sem)
        c1.wait()
        c2.wait()
        c3.wait()
        c4.wait()

        for g in range(_BPW // _L):
            sl = pl.ds(g * _L, _L)
            sv[sl] = wb[sl] + vb[sl]

        lane = lax.iota(jnp.int32, _L)
        dn = lax.GatherDimensionNumbers(
            offset_dims=(), collapsed_slice_dims=(0,), start_index_map=(0,))

        def perm(v, bit):
            idx = (lane ^ bit).reshape(_L, 1)
            return lax.gather(v, idx, dn, (1,),
                              mode=lax.GatherScatterMode.PROMISE_IN_BOUNDS)

        def merge(a, b, bit):
            hi = (lane & bit) != 0
            return jnp.where(hi, b, a) + perm(jnp.where(hi, a, b), bit)

        for g in range(_BPW // _L):
            vs = []
            for r in range(_L):
                j = g * _L + r
                acc = wrows[j, pl.ds(0, _L)] * vrows[j, pl.ds(0, _L)]
                for kk in range(1, _E // _L):
                    sl = pl.ds(kk * _L, _L)
                    acc = acc + wrows[j, sl] * vrows[j, sl]
                vs.append(acc)
            for bit in (1, 2, 4, 8):
                vs = [merge(vs[2 * i], vs[2 * i + 1], bit)
                      for i in range(len(vs) // 2)]
            dotv[pl.ds(g * _L, _L)] = vs[0]

        pltpu.sync_copy(dotv, dot_out.at[pl.ds(base, _BPW)])
        pltpu.sync_copy(sv, s_out.at[pl.ds(base, _BPW)])

    return k(w_data, v_data, w_embed, v_embed, w_bias, v_bias)


def _tc_combine_body(dot_ref, s_ref, lab_ref, out_ref):
    d = dot_ref[...]
    s = s_ref[...]
    lab = lab_ref[...]
    w = jnp.minimum(jnp.exp(_ALPHA * jnp.log(lab * (1.0 / _X_MAX))), 1.0)
    a = d - jnp.log(lab)
    s1 = jnp.sum(w * a * a)
    s2 = jnp.sum(w * a)
    s3 = jnp.sum(w)
    s4 = jnp.sum(s)
    s5 = jnp.sum(s * s)
    bf = float(_B)
    out_ref[0, 0] = (bf * s1 + 2.0 * s2 * s4 + s3 * s5) / (bf * bf)


def _tc_combine(dot, s, labels):
    return pl.pallas_call(
        _tc_combine_body,
        out_shape=jax.ShapeDtypeStruct((1, 1), jnp.float32),
        out_specs=pl.BlockSpec(memory_space=pltpu.SMEM),
    )(dot.reshape(32, 128), s.reshape(32, 128), labels.reshape(32, 128))


def kernel(w_data, v_data, labels, w_embed, w_bias, v_embed, v_bias):
    dot, s = _sc_gather_dot(
        w_data.astype(jnp.int32), v_data.astype(jnp.int32),
        w_embed, v_embed,
        w_bias.reshape(-1), v_bias.reshape(-1),
    )
    out = _tc_combine(dot, s, labels)
    return out[0, 0]
